# Initial kernel scaffold; baseline (speedup 1.0000x reference)
import jax
import jax.numpy as jnp
from jax.experimental import pallas as pl


def kernel(atom, bond, adj_matrix, adj_matrix_tuple, W_atom, b_atom, W_embed, b_embed, W_bond, b_bond):
    raise NotImplementedError("write your pallas kernel here")


# speedup vs baseline: 2.4237x; 2.4237x over previous
import functools

import jax
import jax.numpy as jnp
from jax import lax
from jax.experimental import pallas as pl
from jax.experimental.pallas import tpu as pltpu
from jax.experimental.pallas import tpu_sc as plsc

N_ATOMS = 10000
N_EDGES = 320000
D = 128
F = 16
NC = 2
NS = 16
CHUNK = 128
CHUNKS_PER_WORKER = 80
EDGES_PER_WORKER = CHUNK * CHUNKS_PER_WORKER
E_PAD = EDGES_PER_WORKER * NC * NS
N_PAD = 10240
ROWS_PER_SUBCORE = N_PAD // NS
ZCHUNK = 128

_f32 = jnp.float32


def _h_body(atom_ref, w_ref, b_ref, o_ref):
    o_ref[...] = (
        jnp.dot(atom_ref[...], w_ref[...], preferred_element_type=_f32)
        + b_ref[...]
    )


def _gate_c2_body(bond_ref, w2_ref, gmat_ref, bb_ref, gate_ref, c2_ref):
    b = bond_ref[...]
    gate_ref[...] = jax.nn.sigmoid(jnp.dot(b, gmat_ref[...], preferred_element_type=_f32))
    c2_ref[...] = jnp.dot(b, w2_ref[...], preferred_element_type=_f32) + bb_ref[...]


def _update_body(p_ref, h_ref, we_ref, au_ref, e1_ref, e2_ref):
    au = jax.nn.softplus(p_ref[0] + p_ref[1] + h_ref[...])
    au_ref[...] = au
    we = we_ref[...]
    e1_ref[...] = jnp.dot(au, we[:D], preferred_element_type=_f32)
    e2_ref[...] = jnp.dot(au, we[D:], preferred_element_type=_f32)


def _bond_body(s_ref, c2_ref, wb1_ref, be_ref, o_ref):
    d = jax.nn.softplus(s_ref[...] + be_ref[...])
    o_ref[...] = jax.nn.softplus(
        jnp.dot(d, wb1_ref[...], preferred_element_type=_f32) + c2_ref[...]
    )


_MESH = plsc.VectorSubcoreMesh(core_axis_name="c", subcore_axis_name="s")


def _scatter_kernel(src_hbm, dst_hbm, gate_hbm, h_hbm, out_hbm,
                    src_v, dst_v, gate_v, rows_v, accum, sem):
    cid = lax.axis_index("c")
    sid = lax.axis_index("s")

    @pl.loop(0, ZCHUNK)
    def _(r):
        for k in range(D // 16):
            rows_v[r, pl.ds(k * 16, 16)] = jnp.zeros((16,), _f32)

    zbase = sid * ROWS_PER_SUBCORE
    for t in range(ROWS_PER_SUBCORE // ZCHUNK):
        pltpu.sync_copy(rows_v.at[pl.ds(0, ZCHUNK)],
                        accum.at[pl.ds(zbase + t * ZCHUNK, ZCHUNK)])
    plsc.subcore_barrier()

    base = (cid * NS + sid) * EDGES_PER_WORKER

    @pl.loop(0, CHUNKS_PER_WORKER)
    def _(t):
        off = base + t * CHUNK
        pltpu.sync_copy(src_hbm.at[pl.ds(off, CHUNK)], src_v)
        pltpu.sync_copy(dst_hbm.at[pl.ds(off, CHUNK)], dst_v)
        pltpu.sync_copy(gate_hbm.at[pl.ds(off, CHUNK)], gate_v)
        pltpu.async_copy(h_hbm.at[src_v], rows_v, sem).wait()

        @pl.loop(0, CHUNK // 16)
        def _(grp):
            e0 = grp * 16
            gvec = gate_v[pl.ds(e0, 16)]
            for j in range(16):
                g = gvec[j]
                for k in range(D // 16):
                    sl = pl.ds(k * 16, 16)
                    rows_v[e0 + j, sl] = rows_v[e0 + j, sl] * g

        pltpu.sync_copy(rows_v, accum.at[dst_v], add=True)

    plsc.subcore_barrier()
    for t in range(ROWS_PER_SUBCORE // ZCHUNK):
        r0 = zbase + t * ZCHUNK
        pltpu.sync_copy(accum.at[pl.ds(r0, ZCHUNK)],
                        out_hbm.at[cid, pl.ds(r0, ZCHUNK)])


def _pair_gather_kernel(i_hbm, j_hbm, e1_hbm, e2_hbm, s_hbm,
                        i_v, j_v, r1_v, r2_v, sem1, sem2):
    cid = lax.axis_index("c")
    sid = lax.axis_index("s")
    base = (cid * NS + sid) * EDGES_PER_WORKER

    @pl.loop(0, CHUNKS_PER_WORKER)
    def _(t):
        off = base + t * CHUNK
        pltpu.sync_copy(i_hbm.at[pl.ds(off, CHUNK)], i_v)
        pltpu.sync_copy(j_hbm.at[pl.ds(off, CHUNK)], j_v)
        c1 = pltpu.async_copy(e1_hbm.at[i_v], r1_v, sem1)
        c2 = pltpu.async_copy(e2_hbm.at[j_v], r2_v, sem2)
        c1.wait()
        c2.wait()

        @pl.loop(0, CHUNK)
        def _(e):
            r1_v[e, pl.ds(0, 16)] = r1_v[e, pl.ds(0, 16)] + r2_v[e, pl.ds(0, 16)]

        pltpu.sync_copy(r1_v, s_hbm.at[pl.ds(off, CHUNK)])


_sc_scatter = functools.partial(
    pl.kernel,
    _scatter_kernel,
    out_type=jax.ShapeDtypeStruct((NC, N_PAD, D), _f32),
    mesh=_MESH,
    scratch_types=[
        pltpu.VMEM((CHUNK,), jnp.int32),
        pltpu.VMEM((CHUNK,), jnp.int32),
        pltpu.VMEM((CHUNK,), _f32),
        pltpu.VMEM((CHUNK, D), _f32),
        pltpu.VMEM_SHARED((N_PAD, D), _f32),
        pltpu.SemaphoreType.DMA,
    ],
)


_sc_pair_gather = functools.partial(
    pl.kernel,
    _pair_gather_kernel,
    out_type=jax.ShapeDtypeStruct((E_PAD, F), _f32),
    mesh=_MESH,
    compiler_params=pltpu.CompilerParams(use_tc_tiling_on_sc=False),
    scratch_types=[
        pltpu.VMEM((CHUNK,), jnp.int32),
        pltpu.VMEM((CHUNK,), jnp.int32),
        pltpu.VMEM((CHUNK, F), _f32),
        pltpu.VMEM((CHUNK, F), _f32),
        pltpu.SemaphoreType.DMA,
        pltpu.SemaphoreType.DMA,
    ],
)


def kernel(atom, bond, adj_matrix, adj_matrix_tuple,
           W_atom, b_atom, W_embed, b_embed, W_bond, b_bond):
    src = adj_matrix[0].astype(jnp.int32)
    dst = adj_matrix[1].astype(jnp.int32)
    idx_i = adj_matrix_tuple[:, 0].astype(jnp.int32)
    idx_j = adj_matrix_tuple[:, 1].astype(jnp.int32)

    h = pl.pallas_call(
        _h_body,
        out_shape=jax.ShapeDtypeStruct((N_ATOMS, D), _f32),
    )(atom, W_atom, b_atom)

    eye8 = jnp.eye(8, dtype=_f32)
    w2_blk = jnp.kron(eye8, W_bond[F:])
    wb1_blk = jnp.kron(eye8, W_bond[:F])
    gmat = jnp.kron(eye8, jnp.full((F, 1), 1.0 / F, _f32))
    bb_tile = jnp.tile(b_bond, 8)
    be_tile = jnp.tile(b_embed, 8)
    bond128 = bond.reshape(N_EDGES // 8, 8 * F)

    er = N_EDGES // 8 // 8
    gate8, c2 = pl.pallas_call(
        _gate_c2_body,
        grid=(8,),
        in_specs=[
            pl.BlockSpec((er, 8 * F), lambda i: (i, 0)),
            pl.BlockSpec((8 * F, 8 * F), lambda i: (0, 0)),
            pl.BlockSpec((8 * F, 8), lambda i: (0, 0)),
            pl.BlockSpec((8 * F,), lambda i: (0,)),
        ],
        out_specs=[
            pl.BlockSpec((er, 8), lambda i: (i, 0)),
            pl.BlockSpec((er, 8 * F), lambda i: (i, 0)),
        ],
        out_shape=[
            jax.ShapeDtypeStruct((N_EDGES // 8, 8), _f32),
            jax.ShapeDtypeStruct((N_EDGES // 8, 8 * F), _f32),
        ],
    )(bond128, w2_blk, gmat, bb_tile)
    gate = gate8.reshape(N_EDGES)

    pad = E_PAD - N_EDGES
    src_p = jnp.pad(src, (0, pad))
    dst_p = jnp.pad(dst, (0, pad))
    gate_p = jnp.pad(gate, (0, pad))
    i_p = jnp.pad(idx_i, (0, pad))
    j_p = jnp.pad(idx_j, (0, pad))

    partials = _sc_scatter()(src_p, dst_p, gate_p, h)[:, :N_ATOMS]

    atom_update, e1, e2 = pl.pallas_call(
        _update_body,
        out_shape=[
            jax.ShapeDtypeStruct((N_ATOMS, D), _f32),
            jax.ShapeDtypeStruct((N_ATOMS, F), _f32),
            jax.ShapeDtypeStruct((N_ATOMS, F), _f32),
        ],
    )(partials, h, W_embed)

    s_pad = _sc_pair_gather()(i_p, j_p, e1, e2)
    s128 = s_pad[:N_EDGES].reshape(N_EDGES // 8, 8 * F)

    bond_update = pl.pallas_call(
        _bond_body,
        grid=(8,),
        in_specs=[
            pl.BlockSpec((er, 8 * F), lambda i: (i, 0)),
            pl.BlockSpec((er, 8 * F), lambda i: (i, 0)),
            pl.BlockSpec((8 * F, 8 * F), lambda i: (0, 0)),
            pl.BlockSpec((8 * F,), lambda i: (0,)),
        ],
        out_specs=pl.BlockSpec((er, 8 * F), lambda i: (i, 0)),
        out_shape=jax.ShapeDtypeStruct((N_EDGES // 8, 8 * F), _f32),
    )(s128, c2, wb1_blk, be_tile).reshape(N_EDGES, F)

    return (atom_update, bond_update)


# speedup vs baseline: 6.1150x; 2.5230x over previous
import functools

import jax
import jax.numpy as jnp
from jax import lax
from jax.experimental import pallas as pl
from jax.experimental.pallas import tpu as pltpu
from jax.experimental.pallas import tpu_sc as plsc

N_ATOMS = 10000
N_EDGES = 320000
D = 128
F = 16
NC = 2
NS = 16
CHUNK = 128
CHUNKS_PER_WORKER = 80
EDGES_PER_WORKER = CHUNK * CHUNKS_PER_WORKER
E_PAD = EDGES_PER_WORKER * NC * NS
N_PAD = 10240
ROWS_PER_SUBCORE = N_PAD // NS
ZCHUNK = 128

_f32 = jnp.float32


def _h_body(atom_ref, w_ref, b_ref, o_ref):
    o_ref[...] = (
        jnp.dot(atom_ref[...], w_ref[...], preferred_element_type=_f32)
        + b_ref[...]
    )


def _gate_c2_body(bond_ref, w2_ref, gmat_ref, bb_ref, gate_ref, c2_ref):
    b = bond_ref[...]
    gate_ref[...] = jax.nn.sigmoid(jnp.dot(b, gmat_ref[...], preferred_element_type=_f32))
    c2_ref[...] = jnp.dot(b, w2_ref[...], preferred_element_type=_f32) + bb_ref[...]


def _update_body(p_ref, h_ref, we_ref, au_ref, e1_ref, e2_ref):
    au = jax.nn.softplus(p_ref[0] + p_ref[1] + h_ref[...])
    au_ref[...] = au
    we = we_ref[...]
    e1_ref[...] = jnp.dot(au, we[:D], preferred_element_type=_f32)
    e2_ref[...] = jnp.dot(au, we[D:], preferred_element_type=_f32)


def _bond_body(s_ref, c2_ref, wb1_ref, be_ref, o_ref):
    d = jax.nn.softplus(s_ref[...] + be_ref[...])
    o_ref[...] = jax.nn.softplus(
        jnp.dot(d, wb1_ref[...], preferred_element_type=_f32) + c2_ref[...]
    )


_MESH = plsc.VectorSubcoreMesh(core_axis_name="c", subcore_axis_name="s")


def _scatter_kernel(src_hbm, dst_hbm, gate_hbm, h_hbm, out_hbm,
                    src_a, src_b, dst_a, dst_b, gate_a, gate_b,
                    rows_a, rows_b, accum,
                    isem_a, isem_b, gsem_a, gsem_b):
    cid = lax.axis_index("c")
    sid = lax.axis_index("s")

    @pl.loop(0, ZCHUNK)
    def _(r):
        for k in range(D // 16):
            rows_a[r, pl.ds(k * 16, 16)] = jnp.zeros((16,), _f32)

    zbase = sid * ROWS_PER_SUBCORE
    for t in range(ROWS_PER_SUBCORE // ZCHUNK):
        pltpu.sync_copy(rows_a.at[pl.ds(0, ZCHUNK)],
                        accum.at[pl.ds(zbase + t * ZCHUNK, ZCHUNK)])
    plsc.subcore_barrier()

    wrow = (cid * NS + sid) * CHUNKS_PER_WORKER

    def issue_idx(t, sv, dv, gv, sem):
        pltpu.async_copy(src_hbm.at[pl.ds(wrow + t, 1)], sv, sem)
        pltpu.async_copy(dst_hbm.at[pl.ds(wrow + t, 1)], dv, sem)
        pltpu.async_copy(gate_hbm.at[pl.ds(wrow + t, 1)], gv, sem)

    def wait_idx(sv, dv, gv, sem):
        pltpu.make_async_copy(src_hbm.at[pl.ds(wrow, 1)], sv, sem).wait()
        pltpu.make_async_copy(dst_hbm.at[pl.ds(wrow, 1)], dv, sem).wait()
        pltpu.make_async_copy(gate_hbm.at[pl.ds(wrow, 1)], gv, sem).wait()

    def scale(buf, gv):
        @pl.loop(0, CHUNK // 16)
        def _(grp):
            e0 = grp * 16
            gvec = gv[0, pl.ds(e0, 16)]
            for j in range(16):
                g = gvec[j]
                for k in range(D // 16):
                    sl = pl.ds(k * 16, 16)
                    buf[e0 + j, sl] = buf[e0 + j, sl] * g

    def wait_gather(buf, sem):
        pltpu.make_async_copy(h_hbm.at[src_a.at[0]], buf, sem).wait()

    issue_idx(0, src_a, dst_a, gate_a, isem_a)
    wait_idx(src_a, dst_a, gate_a, isem_a)
    pltpu.async_copy(h_hbm.at[src_a.at[0]], rows_a, gsem_a)
    issue_idx(1, src_b, dst_b, gate_b, isem_b)

    @pl.loop(0, CHUNKS_PER_WORKER - 2, step=2)
    def _(t):
        wait_idx(src_b, dst_b, gate_b, isem_b)
        pltpu.async_copy(h_hbm.at[src_b.at[0]], rows_b, gsem_b)
        wait_gather(rows_a, gsem_a)
        scale(rows_a, gate_a)
        pltpu.sync_copy(rows_a, accum.at[dst_a.at[0]], add=True)
        issue_idx(t + 2, src_a, dst_a, gate_a, isem_a)

        wait_idx(src_a, dst_a, gate_a, isem_a)
        pltpu.async_copy(h_hbm.at[src_a.at[0]], rows_a, gsem_a)
        wait_gather(rows_b, gsem_b)
        scale(rows_b, gate_b)
        pltpu.sync_copy(rows_b, accum.at[dst_b.at[0]], add=True)
        issue_idx(t + 3, src_b, dst_b, gate_b, isem_b)

    wait_idx(src_b, dst_b, gate_b, isem_b)
    pltpu.async_copy(h_hbm.at[src_b.at[0]], rows_b, gsem_b)
    wait_gather(rows_a, gsem_a)
    scale(rows_a, gate_a)
    pltpu.sync_copy(rows_a, accum.at[dst_a.at[0]], add=True)
    wait_gather(rows_b, gsem_b)
    scale(rows_b, gate_b)
    pltpu.sync_copy(rows_b, accum.at[dst_b.at[0]], add=True)

    plsc.subcore_barrier()
    for t in range(ROWS_PER_SUBCORE // ZCHUNK):
        r0 = zbase + t * ZCHUNK
        pltpu.sync_copy(accum.at[pl.ds(r0, ZCHUNK)],
                        out_hbm.at[cid, pl.ds(r0, ZCHUNK)])


def _pair_gather_kernel(i_hbm, j_hbm, e1_hbm, e2_hbm, s_hbm,
                        ivs, jvs, r1a, r2a, r1b, r2b, stage,
                        sem1a, sem2a, sem1b, sem2b):
    cid = lax.axis_index("c")
    sid = lax.axis_index("s")
    wrow = (cid * NS + sid) * CHUNKS_PER_WORKER
    pltpu.sync_copy(i_hbm.at[pl.ds(wrow, CHUNKS_PER_WORKER)], ivs)
    pltpu.sync_copy(j_hbm.at[pl.ds(wrow, CHUNKS_PER_WORKER)], jvs)

    def issue(t, r1, r2, s1, s2):
        pltpu.async_copy(e1_hbm.at[ivs.at[t]], r1, s1)
        pltpu.async_copy(e2_hbm.at[jvs.at[t]], r2, s2)

    def finish(t, r1, r2, s1, s2):
        pltpu.make_async_copy(e1_hbm.at[ivs.at[0]], r1, s1).wait()
        pltpu.make_async_copy(e2_hbm.at[jvs.at[0]], r2, s2).wait()

        @pl.loop(0, CHUNK // 8)
        def _(r):
            for p in range(8):
                e = r * 8 + p
                stage[r, pl.ds(p * F, F)] = (
                    r1[e, pl.ds(0, F)] + r2[e, pl.ds(0, F)]
                )

        pltpu.sync_copy(stage, s_hbm.at[pl.ds((wrow + t) * (CHUNK // 8),
                                              CHUNK // 8)])

    issue(0, r1a, r2a, sem1a, sem2a)

    @pl.loop(0, CHUNKS_PER_WORKER - 2, step=2)
    def _(t):
        issue(t + 1, r1b, r2b, sem1b, sem2b)
        finish(t, r1a, r2a, sem1a, sem2a)
        issue(t + 2, r1a, r2a, sem1a, sem2a)
        finish(t + 1, r1b, r2b, sem1b, sem2b)

    tl = CHUNKS_PER_WORKER - 2
    issue(tl + 1, r1b, r2b, sem1b, sem2b)
    finish(tl, r1a, r2a, sem1a, sem2a)
    finish(tl + 1, r1b, r2b, sem1b, sem2b)


_sc_scatter = functools.partial(
    pl.kernel,
    _scatter_kernel,
    out_type=jax.ShapeDtypeStruct((NC, N_PAD, D), _f32),
    mesh=_MESH,
    scratch_types=[
        pltpu.VMEM((1, CHUNK), jnp.int32),
        pltpu.VMEM((1, CHUNK), jnp.int32),
        pltpu.VMEM((1, CHUNK), jnp.int32),
        pltpu.VMEM((1, CHUNK), jnp.int32),
        pltpu.VMEM((1, CHUNK), _f32),
        pltpu.VMEM((1, CHUNK), _f32),
        pltpu.VMEM((CHUNK, D), _f32),
        pltpu.VMEM((CHUNK, D), _f32),
        pltpu.VMEM_SHARED((N_PAD, D), _f32),
        pltpu.SemaphoreType.DMA,
        pltpu.SemaphoreType.DMA,
        pltpu.SemaphoreType.DMA,
        pltpu.SemaphoreType.DMA,
    ],
)


_sc_pair_gather = functools.partial(
    pl.kernel,
    _pair_gather_kernel,
    out_type=jax.ShapeDtypeStruct((E_PAD // 8, 8 * F), _f32),
    mesh=_MESH,
    compiler_params=pltpu.CompilerParams(use_tc_tiling_on_sc=False),
    scratch_types=[
        pltpu.VMEM((CHUNKS_PER_WORKER, CHUNK), jnp.int32),
        pltpu.VMEM((CHUNKS_PER_WORKER, CHUNK), jnp.int32),
        pltpu.VMEM((CHUNK, F), _f32),
        pltpu.VMEM((CHUNK, F), _f32),
        pltpu.VMEM((CHUNK, F), _f32),
        pltpu.VMEM((CHUNK, F), _f32),
        pltpu.VMEM((CHUNK // 8, 8 * F), _f32),
        pltpu.SemaphoreType.DMA,
        pltpu.SemaphoreType.DMA,
        pltpu.SemaphoreType.DMA,
        pltpu.SemaphoreType.DMA,
    ],
)


def kernel(atom, bond, adj_matrix, adj_matrix_tuple,
           W_atom, b_atom, W_embed, b_embed, W_bond, b_bond):
    src = adj_matrix[0].astype(jnp.int32)
    dst = adj_matrix[1].astype(jnp.int32)
    idx_i = adj_matrix_tuple[:, 0].astype(jnp.int32)
    idx_j = adj_matrix_tuple[:, 1].astype(jnp.int32)

    h = pl.pallas_call(
        _h_body,
        out_shape=jax.ShapeDtypeStruct((N_ATOMS, D), _f32),
    )(atom, W_atom, b_atom)

    eye8 = jnp.eye(8, dtype=_f32)
    w2_blk = jnp.kron(eye8, W_bond[F:])
    wb1_blk = jnp.kron(eye8, W_bond[:F])
    gmat = jnp.kron(eye8, jnp.full((F, 1), 1.0 / F, _f32))
    bb_tile = jnp.tile(b_bond, 8)
    be_tile = jnp.tile(b_embed, 8)
    bond128 = bond.reshape(N_EDGES // 8, 8 * F)

    er = N_EDGES // 8 // 8
    gate8, c2 = pl.pallas_call(
        _gate_c2_body,
        grid=(8,),
        in_specs=[
            pl.BlockSpec((er, 8 * F), lambda i: (i, 0)),
            pl.BlockSpec((8 * F, 8 * F), lambda i: (0, 0)),
            pl.BlockSpec((8 * F, 8), lambda i: (0, 0)),
            pl.BlockSpec((8 * F,), lambda i: (0,)),
        ],
        out_specs=[
            pl.BlockSpec((er, 8), lambda i: (i, 0)),
            pl.BlockSpec((er, 8 * F), lambda i: (i, 0)),
        ],
        out_shape=[
            jax.ShapeDtypeStruct((N_EDGES // 8, 8), _f32),
            jax.ShapeDtypeStruct((N_EDGES // 8, 8 * F), _f32),
        ],
    )(bond128, w2_blk, gmat, bb_tile)
    gate = gate8.reshape(N_EDGES)

    pad = E_PAD - N_EDGES
    spread = jnp.arange(pad, dtype=jnp.int32)
    src_p = jnp.concatenate([src, spread % N_ATOMS])
    dst_p = jnp.concatenate([dst, N_ATOMS + spread % (N_PAD - N_ATOMS)])
    gate_p = jnp.pad(gate, (0, pad))
    i_p = jnp.concatenate([idx_i, spread % N_ATOMS])
    j_p = jnp.concatenate([idx_j, spread % N_ATOMS])
    n_chunks = E_PAD // CHUNK
    to2d = lambda a: a.reshape(n_chunks, CHUNK)

    partials = _sc_scatter()(to2d(src_p), to2d(dst_p), to2d(gate_p), h)

    atom_update, e1, e2 = pl.pallas_call(
        _update_body,
        grid=(1,),
        in_specs=[
            pl.BlockSpec((NC, N_ATOMS, D), lambda i: (0, 0, 0)),
            pl.BlockSpec((N_ATOMS, D), lambda i: (0, 0)),
            pl.BlockSpec((2 * D, F), lambda i: (0, 0)),
        ],
        out_specs=[
            pl.BlockSpec((N_ATOMS, D), lambda i: (0, 0)),
            pl.BlockSpec((N_ATOMS, F), lambda i: (0, 0)),
            pl.BlockSpec((N_ATOMS, F), lambda i: (0, 0)),
        ],
        out_shape=[
            jax.ShapeDtypeStruct((N_ATOMS, D), _f32),
            jax.ShapeDtypeStruct((N_ATOMS, F), _f32),
            jax.ShapeDtypeStruct((N_ATOMS, F), _f32),
        ],
    )(partials, h, W_embed)

    s128 = _sc_pair_gather()(to2d(i_p), to2d(j_p), e1, e2)

    bond_update = pl.pallas_call(
        _bond_body,
        grid=(8,),
        in_specs=[
            pl.BlockSpec((er, 8 * F), lambda i: (i, 0)),
            pl.BlockSpec((er, 8 * F), lambda i: (i, 0)),
            pl.BlockSpec((8 * F, 8 * F), lambda i: (0, 0)),
            pl.BlockSpec((8 * F,), lambda i: (0,)),
        ],
        out_specs=pl.BlockSpec((er, 8 * F), lambda i: (i, 0)),
        out_shape=jax.ShapeDtypeStruct((N_EDGES // 8, 8 * F), _f32),
    )(s128, c2, wb1_blk, be_tile).reshape(N_EDGES, F)

    return (atom_update, bond_update)
